# baseline (speedup 1.0000x reference)
import jax
import jax.numpy as jnp
from jax.experimental import pallas as pl
from jax.experimental.pallas import tpu as pltpu

E = 8
P = 1
D = 768
EPS = 1e-4
TILE = 512


def _body(x_ref, pat_ref, W_ref, b_ref, o_ref):
    x = x_ref[...]
    pats = pat_ref[...]
    logits = jax.lax.dot_general(
        x, pats, (((1,), (1,)), ((), ())),
        preferred_element_type=jnp.float32)
    s = jax.nn.sigmoid(logits)
    sr = s.reshape(s.shape[0], E, P)
    presence = sr[:, :, 0]
    for p in range(1, P):
        presence = presence * sr[:, :, p]
    g = jnp.where(presence > EPS, presence, 0.0)
    acc = jnp.dot(g, b_ref[...], preferred_element_type=jnp.float32)
    xb = x.astype(jnp.bfloat16)
    for e in range(E):
        y = jnp.dot(xb, W_ref[e], preferred_element_type=jnp.float32)
        acc = acc + g[:, e:e + 1] * y
    o_ref[...] = acc


@jax.jit
def kernel(x, patterns, W, b):
    T = x.shape[0]
    pats = patterns.reshape(E * P, D)
    Wb = W.astype(jnp.bfloat16)
    grid = (T // TILE,)
    return pl.pallas_call(
        _body,
        grid=grid,
        in_specs=[
            pl.BlockSpec((TILE, D), lambda i: (i, 0)),
            pl.BlockSpec((E * P, D), lambda i: (0, 0)),
            pl.BlockSpec((E, D, D), lambda i: (0, 0, 0)),
            pl.BlockSpec((E, D), lambda i: (0, 0)),
        ],
        out_specs=pl.BlockSpec((TILE, D), lambda i: (i, 0)),
        out_shape=jax.ShapeDtypeStruct((T, D), x.dtype),
        compiler_params=pltpu.CompilerParams(
            dimension_semantics=("parallel",)),
    )(x, pats, Wb, b)


# speedup vs baseline: 1.0282x; 1.0282x over previous
import jax
import jax.numpy as jnp
from jax.experimental import pallas as pl
from jax.experimental.pallas import tpu as pltpu

E = 8
P = 1
D = 768
EPS = 1e-4
TILE = 2048


def _body(x_ref, pat_ref, W_ref, b_ref, o_ref):
    x = x_ref[...]
    pats = pat_ref[...]
    logits = jax.lax.dot_general(
        x, pats, (((1,), (1,)), ((), ())),
        preferred_element_type=jnp.float32)
    s = jax.nn.sigmoid(logits)
    sr = s.reshape(s.shape[0], E, P)
    presence = sr[:, :, 0]
    for p in range(1, P):
        presence = presence * sr[:, :, p]
    g = jnp.where(presence > EPS, presence, 0.0)
    acc = jnp.dot(g, b_ref[...], preferred_element_type=jnp.float32)
    xb = x.astype(jnp.bfloat16)
    for e in range(E):
        y = jax.lax.dot_general(
            xb, W_ref[e], (((1,), (0,)), ((), ())),
            precision=jax.lax.Precision.DEFAULT,
            preferred_element_type=jnp.float32)
        acc = acc + g[:, e:e + 1] * y
    o_ref[...] = acc


@jax.jit
def kernel(x, patterns, W, b):
    T = x.shape[0]
    pats = patterns.reshape(E * P, D)
    Wb = W.astype(jnp.bfloat16)
    grid = (T // TILE,)
    return pl.pallas_call(
        _body,
        grid=grid,
        in_specs=[
            pl.BlockSpec((TILE, D), lambda i: (i, 0)),
            pl.BlockSpec((E * P, D), lambda i: (0, 0)),
            pl.BlockSpec((E, D, D), lambda i: (0, 0, 0)),
            pl.BlockSpec((E, D), lambda i: (0, 0)),
        ],
        out_specs=pl.BlockSpec((TILE, D), lambda i: (i, 0)),
        out_shape=jax.ShapeDtypeStruct((T, D), x.dtype),
        compiler_params=pltpu.CompilerParams(
            dimension_semantics=("parallel",)),
    )(x, pats, Wb, b)


# speedup vs baseline: 1.0311x; 1.0028x over previous
import jax
import jax.numpy as jnp
from jax.experimental import pallas as pl
from jax.experimental.pallas import tpu as pltpu

E = 8
P = 1
D = 768
EPS = 1e-4
TILE = 1024


def _body(x_ref, pat_ref, W_ref, b_ref, o_ref):
    x = x_ref[...]
    pats = pat_ref[...]
    logits = jax.lax.dot_general(
        x, pats, (((1,), (1,)), ((), ())),
        preferred_element_type=jnp.float32)
    s = jax.nn.sigmoid(logits)
    sr = s.reshape(s.shape[0], E, P)
    presence = sr[:, :, 0]
    for p in range(1, P):
        presence = presence * sr[:, :, p]
    g = jnp.where(presence > EPS, presence, 0.0)
    acc = jnp.dot(g, b_ref[...], preferred_element_type=jnp.float32)
    xb = x.astype(jnp.bfloat16)
    for e in range(E):
        y = jnp.dot(xb, W_ref[e], preferred_element_type=jnp.float32)
        acc = acc + g[:, e:e + 1] * y
    o_ref[...] = acc


@jax.jit
def kernel(x, patterns, W, b):
    T = x.shape[0]
    pats = patterns.reshape(E * P, D)
    Wb = W.astype(jnp.bfloat16)
    grid = (T // TILE,)
    return pl.pallas_call(
        _body,
        grid=grid,
        in_specs=[
            pl.BlockSpec((TILE, D), lambda i: (i, 0)),
            pl.BlockSpec((E * P, D), lambda i: (0, 0)),
            pl.BlockSpec((E, D, D), lambda i: (0, 0, 0)),
            pl.BlockSpec((E, D), lambda i: (0, 0)),
        ],
        out_specs=pl.BlockSpec((TILE, D), lambda i: (i, 0)),
        out_shape=jax.ShapeDtypeStruct((T, D), x.dtype),
        compiler_params=pltpu.CompilerParams(
            dimension_semantics=("parallel",)),
    )(x, pats, Wb, b)
